# Initial kernel scaffold; baseline (speedup 1.0000x reference)
import jax
import jax.numpy as jnp
from jax.experimental import pallas as pl


def kernel(A_indices, A_values, user_emb, item_emb):
    raise NotImplementedError("write your pallas kernel here")


# speedup vs baseline: 4.7344x; 4.7344x over previous
import functools

import jax
import jax.numpy as jnp
from jax import lax
from jax.experimental import pallas as pl
from jax.experimental.pallas import tpu as pltpu
from jax.experimental.pallas import tpu_sc as plsc

NN = 50000
EE = 800000
HH = 32
NC = 2
NS = 16
LL = 16

EPT = EE // NS
CHUNK = 80
NCHUNK = EPT // CHUNK
MBLK = 2000
NMBLK = EPT // MBLK
CPM = MBLK // CHUNK
NPAD = 50176
RPT = NPAD // NS
PIECE = 112
NPIECE = RPT // PIECE


def _body(row_hbm, col_hbm, val_hbm, x0_hbm, mean_hbm, y1_hbm, y2_hbm,
          acc, colb, rowb, valb, gidx, ridx, vbuf, rows, zb, m1, m2, m3, sem):
    c = lax.axis_index("c")
    s = lax.axis_index("s")
    cN = c * NPAD
    ebase = s * EPT
    rbase = s * RPT

    zero16 = jnp.zeros((LL,), jnp.float32)

    def zfill(i, carry):
        zb[i, pl.ds(0, LL)] = zero16
        zb[i, pl.ds(LL, LL)] = zero16
        return carry
    lax.fori_loop(0, PIECE, zfill, 0)

    def zero_acc():
        def zp(p, carry):
            pltpu.sync_copy(zb, acc.at[pl.ds(rbase + p * PIECE, PIECE)])
            return carry
        lax.fori_loop(0, NPIECE, zp, 0)

    def layer(x_hbm, y_hbm):
        zero_acc()
        plsc.subcore_barrier()

        def mblock(m, carry):
            base = ebase + m * MBLK
            pltpu.sync_copy(col_hbm.at[pl.ds(base, MBLK)], colb)
            pltpu.sync_copy(row_hbm.at[pl.ds(base, MBLK)], rowb)
            pltpu.sync_copy(val_hbm.at[pl.ds(base, MBLK)], valb)

            def chunk(k, carry2):
                off = k * CHUNK
                for j in range(CHUNK // LL):
                    sl = pl.ds(j * LL, LL)
                    gidx[sl] = colb[pl.ds(off + j * LL, LL)] + cN
                    ridx[sl] = rowb[pl.ds(off + j * LL, LL)]
                    vbuf[sl] = valb[pl.ds(off + j * LL, LL)]
                pltpu.async_copy(x_hbm.at[gidx], rows, sem).wait()

                def egroup(g, carry3):
                    vv = vbuf[pl.ds(g * LL, LL)]
                    e0 = g * LL
                    for i in range(LL):
                        v = vv[i]
                        e = e0 + i
                        rows[e, pl.ds(0, LL)] = rows[e, pl.ds(0, LL)] * v
                        rows[e, pl.ds(LL, LL)] = rows[e, pl.ds(LL, LL)] * v
                    return carry3
                lax.fori_loop(0, CHUNK // LL, egroup, 0)
                pltpu.sync_copy(rows, acc.at[ridx], add=True)
                return carry2
            lax.fori_loop(0, CPM, chunk, 0)
            return carry
        lax.fori_loop(0, NMBLK, mblock, 0)
        plsc.subcore_barrier()

        if y_hbm is not None:
            def cp(p, carry):
                r0 = rbase + p * PIECE
                pltpu.sync_copy(acc.at[pl.ds(r0, PIECE)],
                                y_hbm.at[pl.ds(cN + r0, PIECE)])
                return carry
            lax.fori_loop(0, NPIECE, cp, 0)
            plsc.subcore_barrier()

    layer(x0_hbm, y1_hbm)
    layer(y1_hbm, y2_hbm)
    layer(y2_hbm, None)

    def piece(p, carry):
        r0 = rbase + p * PIECE
        pltpu.sync_copy(x0_hbm.at[pl.ds(cN + r0, PIECE)], m1)
        pltpu.sync_copy(y1_hbm.at[pl.ds(cN + r0, PIECE)], m2)
        pltpu.sync_copy(y2_hbm.at[pl.ds(cN + r0, PIECE)], m3)
        pltpu.sync_copy(acc.at[pl.ds(r0, PIECE)], zb)

        def mrow(i, carry2):
            for off in (0, LL):
                sl = pl.ds(off, LL)
                zb[i, sl] = (zb[i, sl] + m1[i, sl] + m2[i, sl] + m3[i, sl]) * 0.25
            return carry2
        lax.fori_loop(0, PIECE, mrow, 0)
        pltpu.sync_copy(zb, mean_hbm.at[pl.ds(cN + r0, PIECE)])
        return carry
    lax.fori_loop(0, NPIECE, piece, 0)


@functools.partial(jax.jit, static_argnums=())
def _propagate(row, col, vals, x0):
    f32 = jnp.float32
    run = pl.kernel(
        _body,
        out_type=(
            jax.ShapeDtypeStruct((2 * NPAD, HH), f32),
            jax.ShapeDtypeStruct((2 * NPAD, HH), f32),
            jax.ShapeDtypeStruct((2 * NPAD, HH), f32),
        ),
        mesh=plsc.VectorSubcoreMesh(
            core_axis_name="c", subcore_axis_name="s",
            num_cores=NC, num_subcores=NS),
        scratch_types=[
            pltpu.VMEM_SHARED((NPAD, HH), f32),
            pltpu.VMEM((MBLK,), jnp.int32),
            pltpu.VMEM((MBLK,), jnp.int32),
            pltpu.VMEM((MBLK,), f32),
            pltpu.VMEM((CHUNK,), jnp.int32),
            pltpu.VMEM((CHUNK,), jnp.int32),
            pltpu.VMEM((CHUNK,), f32),
            pltpu.VMEM((CHUNK, HH), f32),
            pltpu.VMEM((PIECE, HH), f32),
            pltpu.VMEM((PIECE, HH), f32),
            pltpu.VMEM((PIECE, HH), f32),
            pltpu.VMEM((PIECE, HH), f32),
            pltpu.SemaphoreType.DMA,
        ],
        compiler_params=pltpu.CompilerParams(use_tc_tiling_on_sc=False),
    )
    return run(row, col, vals, x0)


def kernel(A_indices, A_values, user_emb, item_emb):
    row = A_indices[0].astype(jnp.int32)
    col = A_indices[1].astype(jnp.int32)
    all_emb = jnp.concatenate([user_emb, item_emb], axis=0)
    pad = jnp.zeros((NPAD - NN, HH), jnp.float32)
    x0 = jnp.concatenate([all_emb[:, :HH], pad, all_emb[:, HH:], pad], axis=0)
    mean_flat, _, _ = _propagate(row, col, A_values, x0)
    nu = NN // 2
    user_final = jnp.concatenate(
        [mean_flat[:nu], mean_flat[NPAD:NPAD + nu]], axis=1)
    item_final = jnp.concatenate(
        [mean_flat[nu:NN], mean_flat[NPAD + nu:NPAD + NN]], axis=1)
    return (user_final, item_final)


# speedup vs baseline: 8.0090x; 1.6917x over previous
import functools

import jax
import jax.numpy as jnp
from jax import lax
from jax.experimental import pallas as pl
from jax.experimental.pallas import tpu as pltpu
from jax.experimental.pallas import tpu_sc as plsc

NN = 50000
EE = 800000
HH = 32
NC = 2
NS = 16
LL = 16

EPT = EE // NS
CHUNK = 80
NCHUNK = EPT // CHUNK
MBLK = 2000
NMBLK = EPT // MBLK
CPM = MBLK // CHUNK
NPAD = 50176
RPT = NPAD // NS
PIECE = 56
NPIECE = RPT // PIECE
RING = 5
ITC = RING * CHUNK
NIT = MBLK // ITC


def _body(row_hbm, col_hbm, val_hbm, x0_hbm, mean_hbm, y1_hbm, y2_hbm,
          acc, colb, rowb, valb, gidxs, ridxs, vbufs, rows3, zb, m1, m2, m3,
          sem_g, sem_s):
    c = lax.axis_index("c")
    s = lax.axis_index("s")
    cN = c * NPAD
    ebase = s * EPT
    rbase = s * RPT

    zero16 = jnp.zeros((LL,), jnp.float32)

    def zfill(i, carry):
        zb[i, pl.ds(0, LL)] = zero16
        zb[i, pl.ds(LL, LL)] = zero16
        return carry
    lax.fori_loop(0, PIECE, zfill, 0)

    def zero_acc():
        def zp(p, carry):
            pltpu.sync_copy(zb, acc.at[pl.ds(rbase + p * PIECE, PIECE)])
            return carry
        lax.fori_loop(0, NPIECE, zp, 0)

    def layer(x_hbm, y_hbm):
        zero_acc()
        plsc.subcore_barrier()

        def mblock(m, carry):
            base = ebase + m * MBLK
            pltpu.sync_copy(col_hbm.at[pl.ds(base, MBLK)], colb)
            pltpu.sync_copy(row_hbm.at[pl.ds(base, MBLK)], rowb)
            pltpu.sync_copy(val_hbm.at[pl.ds(base, MBLK)], valb)

            def it5(t, carry2):
                off0 = t * ITC
                for j in range(RING):
                    off = off0 + j * CHUNK
                    for q in range(CHUNK // LL):
                        sl = pl.ds(q * LL, LL)
                        src = pl.ds(off + q * LL, LL)
                        gidxs[j, sl] = colb[src] + cN
                        ridxs[j, sl] = rowb[src]
                        vbufs[j, sl] = valb[src]
                gd = [pltpu.async_copy(x_hbm.at[gidxs.at[j]], rows3.at[j],
                                       sem_g) for j in range(RING)]
                for d in gd:
                    d.wait()
                sd = []
                for j in range(RING):
                    def egroup(g, carry3, j=j):
                        vv = vbufs[j, pl.ds(g * LL, LL)]
                        e0 = g * LL
                        for i in range(LL):
                            v = vv[i]
                            e = e0 + i
                            rows3[j, e, pl.ds(0, LL)] = rows3[j, e, pl.ds(0, LL)] * v
                            rows3[j, e, pl.ds(LL, LL)] = rows3[j, e, pl.ds(LL, LL)] * v
                        return carry3
                    lax.fori_loop(0, CHUNK // LL, egroup, 0)
                    sd.append(pltpu.async_copy(rows3.at[j], acc.at[ridxs.at[j]],
                                               sem_s, add=True))
                for d in sd:
                    d.wait()
                return carry2
            lax.fori_loop(0, NIT, it5, 0)
            return carry
        lax.fori_loop(0, NMBLK, mblock, 0)
        plsc.subcore_barrier()

        if y_hbm is not None:
            def cp(p, carry):
                r0 = rbase + p * PIECE
                pltpu.sync_copy(acc.at[pl.ds(r0, PIECE)],
                                y_hbm.at[pl.ds(cN + r0, PIECE)])
                return carry
            lax.fori_loop(0, NPIECE, cp, 0)
            plsc.subcore_barrier()

    layer(x0_hbm, y1_hbm)
    layer(y1_hbm, y2_hbm)
    layer(y2_hbm, None)

    def piece(p, carry):
        r0 = rbase + p * PIECE
        pltpu.sync_copy(x0_hbm.at[pl.ds(cN + r0, PIECE)], m1)
        pltpu.sync_copy(y1_hbm.at[pl.ds(cN + r0, PIECE)], m2)
        pltpu.sync_copy(y2_hbm.at[pl.ds(cN + r0, PIECE)], m3)
        pltpu.sync_copy(acc.at[pl.ds(r0, PIECE)], zb)

        def mrow(i, carry2):
            for off in (0, LL):
                sl = pl.ds(off, LL)
                zb[i, sl] = (zb[i, sl] + m1[i, sl] + m2[i, sl] + m3[i, sl]) * 0.25
            return carry2
        lax.fori_loop(0, PIECE, mrow, 0)
        pltpu.sync_copy(zb, mean_hbm.at[pl.ds(cN + r0, PIECE)])
        return carry
    lax.fori_loop(0, NPIECE, piece, 0)


@functools.partial(jax.jit, static_argnums=())
def _propagate(row, col, vals, x0):
    f32 = jnp.float32
    run = pl.kernel(
        _body,
        out_type=(
            jax.ShapeDtypeStruct((2 * NPAD, HH), f32),
            jax.ShapeDtypeStruct((2 * NPAD, HH), f32),
            jax.ShapeDtypeStruct((2 * NPAD, HH), f32),
        ),
        mesh=plsc.VectorSubcoreMesh(
            core_axis_name="c", subcore_axis_name="s",
            num_cores=NC, num_subcores=NS),
        scratch_types=[
            pltpu.VMEM_SHARED((NPAD, HH), f32),
            pltpu.VMEM((MBLK,), jnp.int32),
            pltpu.VMEM((MBLK,), jnp.int32),
            pltpu.VMEM((MBLK,), f32),
            pltpu.VMEM((RING, CHUNK), jnp.int32),
            pltpu.VMEM((RING, CHUNK), jnp.int32),
            pltpu.VMEM((RING, CHUNK), f32),
            pltpu.VMEM((RING, CHUNK, HH), f32),
            pltpu.VMEM((PIECE, HH), f32),
            pltpu.VMEM((PIECE, HH), f32),
            pltpu.VMEM((PIECE, HH), f32),
            pltpu.VMEM((PIECE, HH), f32),
            pltpu.SemaphoreType.DMA,
            pltpu.SemaphoreType.DMA,
        ],
        compiler_params=pltpu.CompilerParams(use_tc_tiling_on_sc=False),
    )
    return run(row, col, vals, x0)


def kernel(A_indices, A_values, user_emb, item_emb):
    row = A_indices[0].astype(jnp.int32)
    col = A_indices[1].astype(jnp.int32)
    all_emb = jnp.concatenate([user_emb, item_emb], axis=0)
    pad = jnp.zeros((NPAD - NN, HH), jnp.float32)
    x0 = jnp.concatenate([all_emb[:, :HH], pad, all_emb[:, HH:], pad], axis=0)
    mean_flat, _, _ = _propagate(row, col, A_values, x0)
    nu = NN // 2
    user_final = jnp.concatenate(
        [mean_flat[:nu], mean_flat[NPAD:NPAD + nu]], axis=1)
    item_final = jnp.concatenate(
        [mean_flat[nu:NN], mean_flat[NPAD + nu:NPAD + NN]], axis=1)
    return (user_final, item_final)


# speedup vs baseline: 9.4841x; 1.1842x over previous
import functools

import jax
import jax.numpy as jnp
from jax import lax
from jax.experimental import pallas as pl
from jax.experimental.pallas import tpu as pltpu
from jax.experimental.pallas import tpu_sc as plsc

NN = 50000
EE = 800000
HH = 32
NC = 2
NS = 16
LL = 16

EPT = EE // NS
CHUNK = 80
NCHUNK = EPT // CHUNK
MBLK = 2000
NMBLK = EPT // MBLK
CPM = MBLK // CHUNK
NPAD = 50176
RPT = NPAD // NS
PIECE = 56
NPIECE = RPT // PIECE
RING = 5
ITC = RING * CHUNK
NIT = MBLK // ITC


def _body(row_hbm, col_hbm, val_hbm, x0_hbm, mean_hbm, y1_hbm, y2_hbm,
          acc, colb, rowb, valb, gidxs, ridxs, vbufs, rows3, zb, m1, m2, m3,
          sg0, sg1, sg2, sg3, sg4, ss0, ss1, ss2, ss3, ss4):
    sem_g = [sg0, sg1, sg2, sg3, sg4]
    sem_s = [ss0, ss1, ss2, ss3, ss4]
    c = lax.axis_index("c")
    s = lax.axis_index("s")
    cN = c * NPAD
    ebase = s * EPT
    rbase = s * RPT

    zero16 = jnp.zeros((LL,), jnp.float32)

    def zfill(i, carry):
        zb[i, pl.ds(0, LL)] = zero16
        zb[i, pl.ds(LL, LL)] = zero16
        return carry
    lax.fori_loop(0, PIECE, zfill, 0)

    def zero_acc():
        def zp(p, carry):
            pltpu.sync_copy(zb, acc.at[pl.ds(rbase + p * PIECE, PIECE)])
            return carry
        lax.fori_loop(0, NPIECE, zp, 0)

    def layer(x_hbm, y_hbm):
        zero_acc()
        plsc.subcore_barrier()

        def mblock(m, carry):
            base = ebase + m * MBLK
            pltpu.sync_copy(col_hbm.at[pl.ds(base, MBLK)], colb)
            pltpu.sync_copy(row_hbm.at[pl.ds(base, MBLK)], rowb)
            pltpu.sync_copy(val_hbm.at[pl.ds(base, MBLK)], valb)

            def it5(t, carry2):
                off0 = t * ITC
                not_first = (m + t) > 0
                for j in range(RING):
                    @pl.when(not_first)
                    def _drain(j=j):
                        pltpu.make_async_copy(
                            rows3.at[j], acc.at[ridxs.at[j]],
                            sem_s[j]).wait()
                    off = off0 + j * CHUNK
                    for q in range(CHUNK // LL):
                        sl = pl.ds(q * LL, LL)
                        src = pl.ds(off + q * LL, LL)
                        gidxs[j, sl] = colb[src] + cN
                        ridxs[j, sl] = rowb[src]
                        vbufs[j, sl] = valb[src]
                    pltpu.async_copy(x_hbm.at[gidxs.at[j]], rows3.at[j],
                                     sem_g[j])
                for j in range(RING):
                    pltpu.make_async_copy(x_hbm.at[gidxs.at[j]], rows3.at[j],
                                          sem_g[j]).wait()
                    def egroup(g, carry3, j=j):
                        vv = vbufs[j, pl.ds(g * LL, LL)]
                        e0 = g * LL
                        for i in range(LL):
                            v = vv[i]
                            e = e0 + i
                            rows3[j, e, pl.ds(0, LL)] = rows3[j, e, pl.ds(0, LL)] * v
                            rows3[j, e, pl.ds(LL, LL)] = rows3[j, e, pl.ds(LL, LL)] * v
                        return carry3
                    lax.fori_loop(0, CHUNK // LL, egroup, 0)
                    pltpu.async_copy(rows3.at[j], acc.at[ridxs.at[j]],
                                     sem_s[j], add=True)
                return carry2
            lax.fori_loop(0, NIT, it5, 0)
            return carry
        lax.fori_loop(0, NMBLK, mblock, 0)
        for j in range(RING):
            pltpu.make_async_copy(rows3.at[j], acc.at[ridxs.at[j]],
                                  sem_s[j]).wait()
        plsc.subcore_barrier()

        if y_hbm is not None:
            def cp(p, carry):
                r0 = rbase + p * PIECE
                pltpu.sync_copy(acc.at[pl.ds(r0, PIECE)],
                                y_hbm.at[pl.ds(cN + r0, PIECE)])
                return carry
            lax.fori_loop(0, NPIECE, cp, 0)
            plsc.subcore_barrier()

    layer(x0_hbm, y1_hbm)
    layer(y1_hbm, y2_hbm)
    layer(y2_hbm, None)

    def piece(p, carry):
        r0 = rbase + p * PIECE
        pltpu.sync_copy(x0_hbm.at[pl.ds(cN + r0, PIECE)], m1)
        pltpu.sync_copy(y1_hbm.at[pl.ds(cN + r0, PIECE)], m2)
        pltpu.sync_copy(y2_hbm.at[pl.ds(cN + r0, PIECE)], m3)
        pltpu.sync_copy(acc.at[pl.ds(r0, PIECE)], zb)

        def mrow(i, carry2):
            for off in (0, LL):
                sl = pl.ds(off, LL)
                zb[i, sl] = (zb[i, sl] + m1[i, sl] + m2[i, sl] + m3[i, sl]) * 0.25
            return carry2
        lax.fori_loop(0, PIECE, mrow, 0)
        pltpu.sync_copy(zb, mean_hbm.at[pl.ds(cN + r0, PIECE)])
        return carry
    lax.fori_loop(0, NPIECE, piece, 0)


@functools.partial(jax.jit, static_argnums=())
def _propagate(row, col, vals, x0):
    f32 = jnp.float32
    run = pl.kernel(
        _body,
        out_type=(
            jax.ShapeDtypeStruct((2 * NPAD, HH), f32),
            jax.ShapeDtypeStruct((2 * NPAD, HH), f32),
            jax.ShapeDtypeStruct((2 * NPAD, HH), f32),
        ),
        mesh=plsc.VectorSubcoreMesh(
            core_axis_name="c", subcore_axis_name="s",
            num_cores=NC, num_subcores=NS),
        scratch_types=[
            pltpu.VMEM_SHARED((NPAD, HH), f32),
            pltpu.VMEM((MBLK,), jnp.int32),
            pltpu.VMEM((MBLK,), jnp.int32),
            pltpu.VMEM((MBLK,), f32),
            pltpu.VMEM((RING, CHUNK), jnp.int32),
            pltpu.VMEM((RING, CHUNK), jnp.int32),
            pltpu.VMEM((RING, CHUNK), f32),
            pltpu.VMEM((RING, CHUNK, HH), f32),
            pltpu.VMEM((PIECE, HH), f32),
            pltpu.VMEM((PIECE, HH), f32),
            pltpu.VMEM((PIECE, HH), f32),
            pltpu.VMEM((PIECE, HH), f32),
        ] + [pltpu.SemaphoreType.DMA] * 10,
        compiler_params=pltpu.CompilerParams(use_tc_tiling_on_sc=False),
    )
    return run(row, col, vals, x0)


def kernel(A_indices, A_values, user_emb, item_emb):
    row = A_indices[0].astype(jnp.int32)
    col = A_indices[1].astype(jnp.int32)
    all_emb = jnp.concatenate([user_emb, item_emb], axis=0)
    pad = jnp.zeros((NPAD - NN, HH), jnp.float32)
    x0 = jnp.concatenate([all_emb[:, :HH], pad, all_emb[:, HH:], pad], axis=0)
    mean_flat, _, _ = _propagate(row, col, A_values, x0)
    nu = NN // 2
    user_final = jnp.concatenate(
        [mean_flat[:nu], mean_flat[NPAD:NPAD + nu]], axis=1)
    item_final = jnp.concatenate(
        [mean_flat[nu:NN], mean_flat[NPAD + nu:NPAD + NN]], axis=1)
    return (user_final, item_final)
